# baseline (speedup 1.0000x reference)
import functools

import jax
import jax.numpy as jnp
from jax import lax
from jax.experimental import pallas as pl
from jax.experimental.pallas import tpu as pltpu
from jax.experimental.pallas import tpu_sc as plsc


@functools.lru_cache(maxsize=None)
def _make_sc_pool(B, L, D):
    info = plsc.get_sparse_core_info()
    NC, NS, LN = info.num_cores, info.num_subcores, info.num_lanes
    NW = NC * NS
    assert B % NW == 0
    b_per_w = B // NW
    HALF = L // 2
    assert HALF <= 128 and L % 2 == 0
    NVR = D // LN
    CH = 16
    NBUF = 8
    LOOK = 3
    assert b_per_w % (2 * CH) == 0 and (2 * CH) % NBUF == 0 and HALF % 4 == 0
    n_ch = b_per_w // CH

    mesh = plsc.VectorSubcoreMesh(core_axis_name="c", subcore_axis_name="s")

    @functools.partial(
        pl.kernel,
        mesh=mesh,
        out_type=jax.ShapeDtypeStruct((B, D), jnp.float32),
        scratch_types=[
            pltpu.VMEM((2, CH, 2, HALF), jnp.int32),
            pltpu.VMEM((NBUF, HALF, D), jnp.float32),
            pltpu.VMEM((2, CH, D), jnp.float32),
            pltpu.SemaphoreType.DMA,
            pltpu.SemaphoreType.DMA,
            pltpu.SemaphoreType.DMA,
            pltpu.SemaphoreType.DMA,
            pltpu.SemaphoreType.DMA,
            pltpu.SemaphoreType.DMA,
            pltpu.SemaphoreType.DMA,
            pltpu.SemaphoreType.DMA,
            pltpu.SemaphoreType.DMA,
            pltpu.SemaphoreType.DMA,
            pltpu.SemaphoreType.DMA,
            pltpu.SemaphoreType.DMA,
        ],
    )
    def sc_pool(ids_hbm, emb_hbm, out_hbm, ids_v, rows_v, out_v,
                sg0, sg1, sg2, sg3, sg4, sg5, sg6, sg7,
                so0, so1, si0, si1):
        wid = lax.axis_index("s") * NC + lax.axis_index("c")
        base = wid * b_per_w
        n = b_per_w
        gsems = (sg0, sg1, sg2, sg3, sg4, sg5, sg6, sg7)
        osems = (so0, so1)
        isems = (si0, si1)

        def gather_half(ids_slot, jrow, half, hslot):
            return pltpu.make_async_copy(
                emb_hbm.at[ids_v.at[ids_slot, jrow, half]],
                rows_v.at[hslot], gsems[hslot])

        def gather_row(ids_slot, jrow, j_for_slot):
            return (gather_half(ids_slot, jrow, 0, (2 * j_for_slot) % NBUF),
                    gather_half(ids_slot, jrow, 1,
                                (2 * j_for_slot + 1) % NBUF))

        def ids_copy(slot, c):
            return pltpu.make_async_copy(
                ids_hbm.at[pl.ds(base + c * CH, CH)], ids_v.at[slot],
                isems[slot])

        def out_copy(p, c):
            return pltpu.make_async_copy(
                out_v.at[p], out_hbm.at[pl.ds(base + c * CH, CH)], osems[p])

        ids_copy(0, 0).start()
        ids_copy(0, 0).wait()
        for r in range(LOOK):
            for cp in gather_row(0, r, r):
                cp.start()

        def half_sum(hs, acc):
            def sum_body(l, a):
                return tuple(
                    (a[k]
                     + rows_v[hs, 4 * l, pl.ds(k * LN, LN)]
                     + rows_v[hs, 4 * l + 1, pl.ds(k * LN, LN)])
                    + (rows_v[hs, 4 * l + 2, pl.ds(k * LN, LN)]
                       + rows_v[hs, 4 * l + 3, pl.ds(k * LN, LN)])
                    for k in range(NVR))
            return lax.fori_loop(0, HALF // 4, sum_body, acc)

        def outer(c2, carry):
            for p in (0, 1):
                c = 2 * c2 + p

                @pl.when(c + 1 < n_ch)
                def _():
                    ids_copy(1 - p, c + 1).start()

                @pl.when(c >= 2)
                def _():
                    out_copy(p, c).wait()

                for j in range(CH):
                    i = c * CH + j
                    nj = (j + LOOK) % CH
                    nslot = p if j < CH - LOOK else 1 - p

                    if j == CH - LOOK:
                        @pl.when(c + 1 < n_ch)
                        def _():
                            ids_copy(1 - p, c + 1).wait()

                    @pl.when(i + LOOK < n)
                    def _():
                        for cp in gather_row(nslot, nj, j + LOOK):
                            cp.start()

                    acc = tuple(jnp.zeros((LN,), jnp.float32)
                                for _ in range(NVR))
                    for half, cp in enumerate(gather_row(p, j, j)):
                        cp.wait()
                        acc = half_sum((2 * j + half) % NBUF, acc)
                    for k in range(NVR):
                        out_v[p, j, pl.ds(k * LN, LN)] = acc[k]

                out_copy(p, c).start()
            return carry

        lax.fori_loop(0, n_ch // 2, outer, 0)
        out_copy(0, n_ch - 2).wait()
        out_copy(1, n_ch - 1).wait()

    return sc_pool


@functools.lru_cache(maxsize=None)
def _make_tc_proj(B, D, L):
    BLK = min(B, 1024)
    assert B % BLK == 0
    inv_l = 1.0 / L

    def proj_body(x_ref, w_ref, b_ref, o_ref):
        x = x_ref[...] * inv_l
        y = lax.dot_general(x, w_ref[...], (((1,), (1,)), ((), ())),
                            preferred_element_type=jnp.float32)
        y = y + b_ref[...]
        ss = jnp.sum(y * y, axis=1, keepdims=True)
        norm = jnp.sqrt(ss)
        o_ref[...] = y / jnp.maximum(norm, 1e-12)

    return pl.pallas_call(
        proj_body,
        grid=(B // BLK,),
        in_specs=[
            pl.BlockSpec((BLK, D), lambda i: (i, 0)),
            pl.BlockSpec((D, D), lambda i: (0, 0)),
            pl.BlockSpec((1, D), lambda i: (0, 0)),
        ],
        out_specs=pl.BlockSpec((BLK, D), lambda i: (i, 0)),
        out_shape=jax.ShapeDtypeStruct((B, D), jnp.float32),
    )


def kernel(ids, emb_weight, proj_weight, proj_bias):
    B, L = ids.shape
    V, D = emb_weight.shape
    ids3 = ids.astype(jnp.int32).reshape(B, 2, L // 2)
    x_sum = _make_sc_pool(B, L, D)(ids3, emb_weight)
    return _make_tc_proj(B, D, L)(x_sum, proj_weight, proj_bias.reshape(1, D))


# speedup vs baseline: 1.0097x; 1.0097x over previous
import functools

import jax
import jax.numpy as jnp
from jax import lax
from jax.experimental import pallas as pl
from jax.experimental.pallas import tpu as pltpu
from jax.experimental.pallas import tpu_sc as plsc


@functools.lru_cache(maxsize=None)
def _make_sc_pool(B, L, D):
    info = plsc.get_sparse_core_info()
    NC, NS, LN = info.num_cores, info.num_subcores, info.num_lanes
    NW = NC * NS
    assert B % NW == 0
    b_per_w = B // NW
    HALF = L // 2
    assert HALF <= 128 and L % 2 == 0
    NVR = D // LN
    CH = 16
    NBUF = 4
    LOOK = 3
    assert b_per_w % (2 * CH) == 0 and CH % NBUF == 0 and L % 4 == 0
    n_ch = b_per_w // CH

    mesh = plsc.VectorSubcoreMesh(core_axis_name="c", subcore_axis_name="s")

    @functools.partial(
        pl.kernel,
        mesh=mesh,
        out_type=jax.ShapeDtypeStruct((B, D), jnp.float32),
        scratch_types=[
            pltpu.VMEM((2, CH, 2, HALF), jnp.int32),
            pltpu.VMEM((NBUF, L, D), jnp.float32),
            pltpu.VMEM((2, CH, D), jnp.float32),
            pltpu.SemaphoreType.DMA,
            pltpu.SemaphoreType.DMA,
            pltpu.SemaphoreType.DMA,
            pltpu.SemaphoreType.DMA,
            pltpu.SemaphoreType.DMA,
            pltpu.SemaphoreType.DMA,
            pltpu.SemaphoreType.DMA,
            pltpu.SemaphoreType.DMA,
        ],
    )
    def sc_pool(ids_hbm, emb_hbm, out_hbm, ids_v, rows_v, out_v,
                sg0, sg1, sg2, sg3, so0, so1, si0, si1):
        wid = lax.axis_index("s") * NC + lax.axis_index("c")
        base = wid * b_per_w
        n = b_per_w
        gsems = (sg0, sg1, sg2, sg3)
        osems = (so0, so1)
        isems = (si0, si1)

        def gather_row(ids_slot, jrow, rslot):
            sem = gsems[rslot]
            c0 = pltpu.make_async_copy(
                emb_hbm.at[ids_v.at[ids_slot, jrow, 0]],
                rows_v.at[rslot, pl.ds(0, HALF)], sem)
            c1 = pltpu.make_async_copy(
                emb_hbm.at[ids_v.at[ids_slot, jrow, 1]],
                rows_v.at[rslot, pl.ds(HALF, HALF)], sem)
            return c0, c1

        def ids_copy(slot, c):
            return pltpu.make_async_copy(
                ids_hbm.at[pl.ds(base + c * CH, CH)], ids_v.at[slot],
                isems[slot])

        def out_copy(p, c):
            return pltpu.make_async_copy(
                out_v.at[p], out_hbm.at[pl.ds(base + c * CH, CH)], osems[p])

        ids_copy(0, 0).start()
        ids_copy(0, 0).wait()
        for r in range(LOOK):
            for cp in gather_row(0, r, r):
                cp.start()

        def outer(c2, carry):
            for p in (0, 1):
                c = 2 * c2 + p

                @pl.when(c + 1 < n_ch)
                def _():
                    ids_copy(1 - p, c + 1).start()

                @pl.when(c >= 2)
                def _():
                    out_copy(p, c).wait()

                for j in range(CH):
                    i = c * CH + j
                    rs = j % NBUF
                    ns = (j + LOOK) % NBUF
                    nj = (j + LOOK) % CH
                    nslot = p if j < CH - LOOK else 1 - p

                    if j == CH - LOOK:
                        @pl.when(c + 1 < n_ch)
                        def _():
                            ids_copy(1 - p, c + 1).wait()

                    @pl.when(i + LOOK < n)
                    def _():
                        for cp in gather_row(nslot, nj, ns):
                            cp.start()

                    for cp in gather_row(p, j, rs):
                        cp.wait()

                    def sum_body(l, acc):
                        return tuple(
                            (acc[k]
                             + rows_v[rs, 4 * l, pl.ds(k * LN, LN)]
                             + rows_v[rs, 4 * l + 1, pl.ds(k * LN, LN)])
                            + (rows_v[rs, 4 * l + 2, pl.ds(k * LN, LN)]
                               + rows_v[rs, 4 * l + 3, pl.ds(k * LN, LN)])
                            for k in range(NVR))

                    acc = lax.fori_loop(
                        0, L // 4, sum_body,
                        tuple(jnp.zeros((LN,), jnp.float32)
                              for _ in range(NVR)))
                    for k in range(NVR):
                        out_v[p, j, pl.ds(k * LN, LN)] = acc[k]

                out_copy(p, c).start()
            return carry

        lax.fori_loop(0, n_ch // 2, outer, 0)
        out_copy(0, n_ch - 2).wait()
        out_copy(1, n_ch - 1).wait()

    return sc_pool


@functools.lru_cache(maxsize=None)
def _make_tc_proj(B, D, L):
    BLK = min(B, 1024)
    assert B % BLK == 0
    inv_l = 1.0 / L

    def proj_body(x_ref, w_ref, b_ref, o_ref):
        x = x_ref[...] * inv_l
        y = lax.dot_general(x, w_ref[...], (((1,), (1,)), ((), ())),
                            preferred_element_type=jnp.float32)
        y = y + b_ref[...]
        ss = jnp.sum(y * y, axis=1, keepdims=True)
        norm = jnp.sqrt(ss)
        o_ref[...] = y / jnp.maximum(norm, 1e-12)

    return pl.pallas_call(
        proj_body,
        grid=(B // BLK,),
        in_specs=[
            pl.BlockSpec((BLK, D), lambda i: (i, 0)),
            pl.BlockSpec((D, D), lambda i: (0, 0)),
            pl.BlockSpec((1, D), lambda i: (0, 0)),
        ],
        out_specs=pl.BlockSpec((BLK, D), lambda i: (i, 0)),
        out_shape=jax.ShapeDtypeStruct((B, D), jnp.float32),
    )


def kernel(ids, emb_weight, proj_weight, proj_bias):
    B, L = ids.shape
    V, D = emb_weight.shape
    ids3 = ids.astype(jnp.int32).reshape(B, 2, L // 2)
    x_sum = _make_sc_pool(B, L, D)(ids3, emb_weight)
    return _make_tc_proj(B, D, L)(x_sum, proj_weight, proj_bias.reshape(1, D))
